# baseline (speedup 1.0000x reference)
import jax
import jax.numpy as jnp
from jax.experimental import pallas as pl
from jax.experimental.pallas import tpu as pltpu

D = 512
K = 1024
Q = 12
B = 512


def _prep_kernel(cb_ref, cc_ref, cb3_ref):
    cb = cb_ref[...]
    cc_ref[...] = jnp.sum(cb * cb, axis=-1, keepdims=True).transpose(0, 2, 1)
    hi = cb.astype(jnp.bfloat16)
    e1 = cb - hi.astype(jnp.float32)
    mid = e1.astype(jnp.bfloat16)
    lo = (e1 - mid.astype(jnp.float32)).astype(jnp.bfloat16)
    cb3_ref[...] = jnp.concatenate([hi, mid, lo], axis=-1)


def _rvq_kernel(x_ref, cb3_ref, cc_ref, out_ref):
    h = x_ref.shape[0] // 2
    rs = [x_ref[:h], x_ref[h:]]
    iota = jax.lax.broadcasted_iota(jnp.int32, (h, K), 1)
    cols = [[], []]
    for q in range(Q):
        hi = cb3_ref[q][:, :D]
        for c in range(2):
            r = rs[c]
            rr = jnp.sum(r * r, axis=1, keepdims=True)
            s2 = jax.lax.dot_general(
                (r + r).astype(jnp.bfloat16), hi,
                (((1,), (1,)), ((), ())),
                preferred_element_type=jnp.float32)
            dist = rr - s2 + cc_ref[q]
            mn = jnp.min(dist, axis=1, keepdims=True)
            idx = jnp.min(jnp.where(dist == mn, iota, K), axis=1)
            cols[c].append(idx)
            if q < Q - 1:
                oh = (iota == idx[:, None]).astype(jnp.bfloat16)
                g = jax.lax.dot_general(
                    oh, cb3_ref[q], (((1,), (0,)), ((), ())),
                    preferred_element_type=jnp.float32)
                rs[c] = r - (g[:, :D] + g[:, D:2 * D] + g[:, 2 * D:])
    out_ref[...] = jnp.concatenate(
        [jnp.stack(cols[0], axis=-1), jnp.stack(cols[1], axis=-1)], axis=0)


def kernel(embedding, codebooks):
    n = embedding.shape[0]
    cc, cb3 = pl.pallas_call(
        _prep_kernel,
        grid=(Q,),
        in_specs=[pl.BlockSpec((1, K, D), lambda q: (q, 0, 0))],
        out_specs=[
            pl.BlockSpec((1, 1, K), lambda q: (q, 0, 0)),
            pl.BlockSpec((1, K, 3 * D), lambda q: (q, 0, 0)),
        ],
        out_shape=[
            jax.ShapeDtypeStruct((Q, 1, K), jnp.float32),
            jax.ShapeDtypeStruct((Q, K, 3 * D), jnp.bfloat16),
        ],
    )(codebooks)
    out = pl.pallas_call(
        _rvq_kernel,
        grid=(n // B,),
        in_specs=[
            pl.BlockSpec((B, D), lambda i: (i, 0)),
            pl.BlockSpec((Q, K, 3 * D), lambda i: (0, 0, 0)),
            pl.BlockSpec((Q, 1, K), lambda i: (0, 0, 0)),
        ],
        out_specs=pl.BlockSpec((B, Q), lambda i: (i, 0)),
        out_shape=jax.ShapeDtypeStruct((n, Q), jnp.int32),
        compiler_params=pltpu.CompilerParams(
            dimension_semantics=("parallel",)),
    )(embedding, cb3, cc)
    return out[:, :, None]


# speedup vs baseline: 1.0729x; 1.0729x over previous
import jax
import jax.numpy as jnp
from jax.experimental import pallas as pl
from jax.experimental.pallas import tpu as pltpu

D = 512
K = 1024
Q = 12
H = 512
B = 2 * H


def _prep_kernel(cb_ref, cc_ref, cb3_ref):
    cb = cb_ref[...]
    cc_ref[...] = jnp.sum(cb * cb, axis=-1, keepdims=True).transpose(0, 2, 1)
    hi = cb.astype(jnp.bfloat16)
    e1 = cb - hi.astype(jnp.float32)
    mid = e1.astype(jnp.bfloat16)
    lo = (e1 - mid.astype(jnp.float32)).astype(jnp.bfloat16)
    cb3_ref[...] = jnp.concatenate([hi, mid, lo], axis=-1)


def _rvq_kernel(x_ref, cb3_ref, cc_ref, out_ref):
    iota = jax.lax.broadcasted_iota(jnp.int32, (H, K), 1)

    def stage(q, r, with_gather):
        cb3 = cb3_ref[q]
        rr = jnp.sum(r * r, axis=1, keepdims=True)
        s2 = jax.lax.dot_general(
            (r + r).astype(jnp.bfloat16), cb3[:, :D],
            (((1,), (1,)), ((), ())),
            preferred_element_type=jnp.float32)
        dist = rr - s2 + cc_ref[q]
        mn = jnp.min(dist, axis=1, keepdims=True)
        idx = jnp.min(jnp.where(dist == mn, iota, K), axis=1)
        if not with_gather:
            return idx, r
        oh = (iota == idx[:, None]).astype(jnp.bfloat16)
        g = jax.lax.dot_general(
            oh, cb3, (((1,), (0,)), ((), ())),
            preferred_element_type=jnp.float32)
        return idx, r - (g[:, :D] + g[:, D:2 * D] + g[:, 2 * D:])

    def body(q, rs):
        r0, r1 = rs
        idx0, r0 = stage(q, r0, True)
        idx1, r1 = stage(q, r1, True)
        out_ref[pl.ds(q, 1), :, pl.ds(0, H)] = idx0[None, None, :]
        out_ref[pl.ds(q, 1), :, pl.ds(H, H)] = idx1[None, None, :]
        return r0, r1

    r0, r1 = jax.lax.fori_loop(
        0, Q - 1, body, (x_ref[:H], x_ref[H:]), unroll=False)
    idx0, _ = stage(Q - 1, r0, False)
    idx1, _ = stage(Q - 1, r1, False)
    out_ref[pl.ds(Q - 1, 1), :, pl.ds(0, H)] = idx0[None, None, :]
    out_ref[pl.ds(Q - 1, 1), :, pl.ds(H, H)] = idx1[None, None, :]


def kernel(embedding, codebooks):
    n = embedding.shape[0]
    cc, cb3 = pl.pallas_call(
        _prep_kernel,
        grid=(Q,),
        in_specs=[pl.BlockSpec((1, K, D), lambda q: (q, 0, 0))],
        out_specs=[
            pl.BlockSpec((1, 1, K), lambda q: (q, 0, 0)),
            pl.BlockSpec((1, K, 3 * D), lambda q: (q, 0, 0)),
        ],
        out_shape=[
            jax.ShapeDtypeStruct((Q, 1, K), jnp.float32),
            jax.ShapeDtypeStruct((Q, K, 3 * D), jnp.bfloat16),
        ],
    )(codebooks)
    out = pl.pallas_call(
        _rvq_kernel,
        grid=(n // B,),
        in_specs=[
            pl.BlockSpec((B, D), lambda i: (i, 0)),
            pl.BlockSpec((Q, K, 3 * D), lambda i: (0, 0, 0)),
            pl.BlockSpec((Q, 1, K), lambda i: (0, 0, 0)),
        ],
        out_specs=pl.BlockSpec((Q, 1, B), lambda i: (0, 0, i)),
        out_shape=jax.ShapeDtypeStruct((Q, 1, n), jnp.int32),
        compiler_params=pltpu.CompilerParams(
            dimension_semantics=("parallel",)),
    )(embedding, cb3, cc)
    return jnp.transpose(out, (2, 0, 1))


# speedup vs baseline: 1.2231x; 1.1400x over previous
import jax
import jax.numpy as jnp
from jax.experimental import pallas as pl
from jax.experimental.pallas import tpu as pltpu

D = 512
K = 1024
Q = 12
B = 512


def _prep_kernel(cb_ref, cc_ref, cb3_ref):
    cb = cb_ref[...]
    cc_ref[...] = jnp.sum(cb * cb, axis=-1, keepdims=True).transpose(0, 2, 1)
    hi = cb.astype(jnp.bfloat16)
    e1 = cb - hi.astype(jnp.float32)
    mid = e1.astype(jnp.bfloat16)
    lo = (e1 - mid.astype(jnp.float32)).astype(jnp.bfloat16)
    cb3_ref[...] = jnp.concatenate([hi, mid, lo], axis=-1)


CHUNK = 256


def _rvq_kernel(x_ref, cb3_ref, cc_ref, out_ref):
    r = x_ref[...]
    nrows = r.shape[0]
    iota = jax.lax.broadcasted_iota(jnp.int32, (nrows, K), 1)
    cols = []
    for q in range(Q):
        rr = jnp.sum(r * r, axis=1, keepdims=True)
        rbf = (r + r).astype(jnp.bfloat16)
        acc_v = None
        for m in range(K // CHUNK):
            lo_k, hi_k = m * CHUNK, (m + 1) * CHUNK
            s2m = jax.lax.dot_general(
                rbf, cb3_ref[q][lo_k:hi_k, :D],
                (((1,), (1,)), ((), ())),
                preferred_element_type=jnp.float32)
            dm = rr - s2m + cc_ref[q][:, lo_k:hi_k]
            im = iota[:, lo_k:hi_k]
            if acc_v is None:
                acc_v, acc_i = dm, im
            else:
                take = dm < acc_v
                acc_v = jnp.where(take, dm, acc_v)
                acc_i = jnp.where(take, im, acc_i)
        mn = jnp.min(acc_v, axis=1, keepdims=True)
        idx = jnp.min(jnp.where(acc_v == mn, acc_i, K), axis=1)
        cols.append(idx)
        if q < Q - 1:
            oh = (iota == idx[:, None]).astype(jnp.bfloat16)
            g = jax.lax.dot_general(
                oh, cb3_ref[q], (((1,), (0,)), ((), ())),
                preferred_element_type=jnp.float32)
            quant = g[:, :D] + g[:, D:2 * D] + g[:, 2 * D:]
            r = r - quant
    out_ref[...] = jnp.stack(cols, axis=-1)


def kernel(embedding, codebooks):
    n = embedding.shape[0]
    cc, cb3 = pl.pallas_call(
        _prep_kernel,
        grid=(Q,),
        in_specs=[pl.BlockSpec((1, K, D), lambda q: (q, 0, 0))],
        out_specs=[
            pl.BlockSpec((1, 1, K), lambda q: (q, 0, 0)),
            pl.BlockSpec((1, K, 3 * D), lambda q: (q, 0, 0)),
        ],
        out_shape=[
            jax.ShapeDtypeStruct((Q, 1, K), jnp.float32),
            jax.ShapeDtypeStruct((Q, K, 3 * D), jnp.bfloat16),
        ],
    )(codebooks)
    out = pl.pallas_call(
        _rvq_kernel,
        grid=(n // B,),
        in_specs=[
            pl.BlockSpec((B, D), lambda i: (i, 0)),
            pl.BlockSpec((Q, K, 3 * D), lambda i: (0, 0, 0)),
            pl.BlockSpec((Q, 1, K), lambda i: (0, 0, 0)),
        ],
        out_specs=pl.BlockSpec((B, Q), lambda i: (i, 0)),
        out_shape=jax.ShapeDtypeStruct((n, Q), jnp.int32),
        compiler_params=pltpu.CompilerParams(
            dimension_semantics=("parallel",)),
    )(embedding, cb3, cc)
    return out[:, :, None]
